# baseline (speedup 1.0000x reference)
import functools

import jax
import jax.numpy as jnp
from jax import lax
from jax.experimental import pallas as pl
from jax.experimental.pallas import tpu as pltpu
from jax.experimental.pallas import tpu_sc as plsc

_N_ATOMS = 320000
_N_MOLS = 10000
_FEAT = 128
_HID = 64

_NC = 2
_NS = 16
_NW = _NC * _NS
_ATOMS_PER_TILE = _N_ATOMS // _NW
_CHUNK = 128
_N_FULL = 78
_TAIL = 16
_STRIPE = 624

_LOG2 = 0.6931471805599453


@functools.partial(
    pl.kernel,
    mesh=plsc.VectorSubcoreMesh(core_axis_name="c", subcore_axis_name="s"),
    out_type=jax.ShapeDtypeStruct((_NC, _N_MOLS, _FEAT), jnp.float32),
    scratch_types=[
        pltpu.VMEM((3, _CHUNK), jnp.int32),
        pltpu.VMEM((_TAIL,), jnp.int32),
        pltpu.VMEM((3, _CHUNK, _FEAT), jnp.float32),
        pltpu.VMEM_SHARED((_N_MOLS, _FEAT), jnp.float32),
        pltpu.SemaphoreType.DMA,
        pltpu.SemaphoreType.DMA,
        pltpu.SemaphoreType.DMA,
        pltpu.SemaphoreType.DMA,
        pltpu.SemaphoreType.DMA,
        pltpu.SemaphoreType.DMA,
        pltpu.SemaphoreType.DMA,
        pltpu.SemaphoreType.DMA,
        pltpu.SemaphoreType.DMA,
    ],
)
def _segsum_sc(feats_hbm, ids_hbm, out_hbm, ibufs, ibuf_t, bufs, acc_sh,
               fsem0, fsem1, fsem2, isem0, isem1, isem2, ssem0, ssem1, ssem2):
    c = lax.axis_index("c")
    s = lax.axis_index("s")
    w = c * _NS + s
    tile_base = w * _ATOMS_PER_TILE
    fsems = (fsem0, fsem1, fsem2)
    isems = (isem0, isem1, isem2)
    ssems = (ssem0, ssem1, ssem2)

    zeros16 = jnp.zeros((16,), jnp.float32)

    def zb(i, carry):
        bufs[0, i // (_FEAT // 16), pl.ds((i % (_FEAT // 16)) * 16, 16)] = zeros16
        return carry

    lax.fori_loop(0, _CHUNK * (_FEAT // 16), zb, 0)

    mol_base = s * _STRIPE

    @pl.when(s < _NS - 1)
    def _():
        for k in range(4):
            pltpu.sync_copy(bufs.at[0],
                            acc_sh.at[pl.ds(mol_base + k * _CHUNK, _CHUNK)])
        pltpu.sync_copy(bufs.at[0].at[pl.ds(0, _STRIPE - 4 * _CHUNK)],
                        acc_sh.at[pl.ds(mol_base + 4 * _CHUNK,
                                        _STRIPE - 4 * _CHUNK)])

    @pl.when(s == _NS - 1)
    def _():
        for k in range(5):
            pltpu.sync_copy(bufs.at[0],
                            acc_sh.at[pl.ds(mol_base + k * _CHUNK, _CHUNK)])

    def start_load(i, b):
        a = tile_base + i * _CHUNK
        pltpu.async_copy(ids_hbm.at[pl.ds(a, _CHUNK)], ibufs.at[b], isems[b])
        pltpu.async_copy(feats_hbm.at[pl.ds(a, _CHUNK)], bufs.at[b], fsems[b])

    def wait_load(i, b):
        a = tile_base + i * _CHUNK
        pltpu.make_async_copy(ids_hbm.at[pl.ds(a, _CHUNK)], ibufs.at[b],
                              isems[b]).wait()
        pltpu.make_async_copy(feats_hbm.at[pl.ds(a, _CHUNK)], bufs.at[b],
                              fsems[b]).wait()

    def start_scatter(b):
        pltpu.async_copy(bufs.at[b], acc_sh.at[ibufs.at[b]], ssems[b], add=True)

    def wait_scatter(b):
        pltpu.make_async_copy(bufs.at[b], acc_sh.at[ibufs.at[b]],
                              ssems[b]).wait()

    for l in range(3):
        start_load(l, l)
    plsc.subcore_barrier()

    def body(j0, carry):
        for b in range(3):
            j = j0 * 3 + b
            wait_load(j, b)

            @pl.when(j >= 2)
            def _():
                wait_scatter((b - 2) % 3)

            start_scatter(b)
            nxt = j + 1

            @pl.when((nxt >= 3) & (nxt < _N_FULL))
            def _():
                start_load(nxt, (b + 1) % 3)

        return carry

    lax.fori_loop(0, _N_FULL // 3, body, 0)

    a_t = tile_base + _N_FULL * _CHUNK
    pltpu.async_copy(ids_hbm.at[pl.ds(a_t, _TAIL)], ibuf_t, isem0)
    pltpu.async_copy(feats_hbm.at[pl.ds(a_t, _TAIL)],
                     bufs.at[0].at[pl.ds(0, _TAIL)], fsem0)
    pltpu.make_async_copy(ids_hbm.at[pl.ds(a_t, _TAIL)], ibuf_t, isem0).wait()
    pltpu.make_async_copy(feats_hbm.at[pl.ds(a_t, _TAIL)],
                          bufs.at[0].at[pl.ds(0, _TAIL)], fsem0).wait()
    wait_scatter(1)
    pltpu.async_copy(bufs.at[0].at[pl.ds(0, _TAIL)], acc_sh.at[ibuf_t],
                     ssem0, add=True)
    wait_scatter(2)
    pltpu.make_async_copy(bufs.at[0].at[pl.ds(0, _TAIL)], acc_sh.at[ibuf_t],
                          ssem0).wait()
    plsc.subcore_barrier()

    @pl.when(s < _NS - 1)
    def _():
        pltpu.sync_copy(acc_sh.at[pl.ds(mol_base, _STRIPE)],
                        out_hbm.at[c, pl.ds(mol_base, _STRIPE)])

    @pl.when(s == _NS - 1)
    def _():
        pltpu.sync_copy(acc_sh.at[pl.ds(mol_base, _STRIPE + _TAIL)],
                        out_hbm.at[c, pl.ds(mol_base, _STRIPE + _TAIL)])


_ROWS_BLK = 1000


def _mlp_body(p_ref, w1_ref, b1_ref, w2_ref, b2_ref, out_ref, fp_ref):
    fp = p_ref[0] + p_ref[1]
    fp_ref[...] = fp
    h = jnp.dot(fp, w1_ref[...], preferred_element_type=jnp.float32) + b1_ref[...]
    sp = jnp.maximum(h, 0.0) + jnp.log(1.0 + jnp.exp(-jnp.abs(h))) - _LOG2
    out_ref[...] = jnp.sum(sp * w2_ref[...], axis=1, keepdims=True) + b2_ref[...]


def _mlp_tc(partials, W1, b1r, w2r, b2r):
    grid = (_N_MOLS // _ROWS_BLK,)
    return pl.pallas_call(
        _mlp_body,
        grid=grid,
        in_specs=[
            pl.BlockSpec((_NC, _ROWS_BLK, _FEAT), lambda i: (0, i, 0)),
            pl.BlockSpec((_FEAT, _HID), lambda i: (0, 0)),
            pl.BlockSpec((1, _HID), lambda i: (0, 0)),
            pl.BlockSpec((1, _HID), lambda i: (0, 0)),
            pl.BlockSpec((1, 1), lambda i: (0, 0)),
        ],
        out_specs=[
            pl.BlockSpec((_ROWS_BLK, 1), lambda i: (i, 0)),
            pl.BlockSpec((_ROWS_BLK, _FEAT), lambda i: (i, 0)),
        ],
        out_shape=[
            jax.ShapeDtypeStruct((_N_MOLS, 1), jnp.float32),
            jax.ShapeDtypeStruct((_N_MOLS, _FEAT), jnp.float32),
        ],
    )(partials, W1, b1r, w2r, b2r)


def kernel(feats, segment_ids, W1, b1, W2, b2):
    ids = segment_ids.astype(jnp.int32)
    partials = _segsum_sc(feats, ids)
    out2d, mol_fp = _mlp_tc(partials, W1, b1.reshape(1, _HID),
                            W2.reshape(1, _HID), b2.reshape(1, 1))
    return out2d.reshape(-1), mol_fp


# speedup vs baseline: 1.1690x; 1.1690x over previous
import functools

import jax
import jax.numpy as jnp
from jax import lax
from jax.experimental import pallas as pl
from jax.experimental.pallas import tpu as pltpu
from jax.experimental.pallas import tpu_sc as plsc

_N_ATOMS = 320000
_N_MOLS = 10000
_FEAT = 128
_HID = 64

_NC = 2
_NS = 16
_NW = _NC * _NS
_ATOMS_PER_TILE = _N_ATOMS // _NW
_CHUNK = 80
_N_CHUNKS = _ATOMS_PER_TILE // _CHUNK
_MOLS_PAD = 10240
_MOLS_PER_TILE = _MOLS_PAD // _NS

_LOG2 = 0.6931471805599453


@functools.partial(
    pl.kernel,
    mesh=plsc.VectorSubcoreMesh(core_axis_name="c", subcore_axis_name="s"),
    out_type=jax.ShapeDtypeStruct((_NC, _MOLS_PAD, _FEAT), jnp.float32),
    scratch_types=[
        pltpu.VMEM((4, _CHUNK), jnp.int32),
        pltpu.VMEM((4, _CHUNK, _FEAT), jnp.float32),
        pltpu.VMEM_SHARED((_MOLS_PAD, _FEAT), jnp.float32),
        pltpu.SemaphoreType.DMA,
        pltpu.SemaphoreType.DMA,
        pltpu.SemaphoreType.DMA,
        pltpu.SemaphoreType.DMA,
        pltpu.SemaphoreType.DMA,
        pltpu.SemaphoreType.DMA,
        pltpu.SemaphoreType.DMA,
        pltpu.SemaphoreType.DMA,
        pltpu.SemaphoreType.DMA,
        pltpu.SemaphoreType.DMA,
        pltpu.SemaphoreType.DMA,
        pltpu.SemaphoreType.DMA,
    ],
)
def _segsum_sc(feats_hbm, ids_hbm, out_hbm, ibufs, bufs, acc_sh,
               fsem0, fsem1, fsem2, fsem3, isem0, isem1, isem2, isem3,
               ssem0, ssem1, ssem2, ssem3):
    c = lax.axis_index("c")
    s = lax.axis_index("s")
    w = c * _NS + s
    tile_base = w * _ATOMS_PER_TILE
    fsems = (fsem0, fsem1, fsem2, fsem3)
    isems = (isem0, isem1, isem2, isem3)
    ssems = (ssem0, ssem1, ssem2, ssem3)

    zeros16 = jnp.zeros((16,), jnp.float32)

    def zb(i, carry):
        bufs[0, i // (_FEAT // 16), pl.ds((i % (_FEAT // 16)) * 16, 16)] = zeros16
        return carry

    lax.fori_loop(0, _CHUNK * (_FEAT // 16), zb, 0)

    mol_base = s * _MOLS_PER_TILE
    for j in range(_MOLS_PER_TILE // _CHUNK):
        pltpu.sync_copy(bufs.at[0], acc_sh.at[pl.ds(mol_base + j * _CHUNK, _CHUNK)])
    plsc.subcore_barrier()

    def start_load(i, b):
        a = tile_base + i * _CHUNK
        pltpu.async_copy(ids_hbm.at[pl.ds(a, _CHUNK)], ibufs.at[b], isems[b])
        pltpu.async_copy(feats_hbm.at[pl.ds(a, _CHUNK)], bufs.at[b], fsems[b])

    def wait_load(i, b):
        a = tile_base + i * _CHUNK
        pltpu.make_async_copy(ids_hbm.at[pl.ds(a, _CHUNK)], ibufs.at[b],
                              isems[b]).wait()
        pltpu.make_async_copy(feats_hbm.at[pl.ds(a, _CHUNK)], bufs.at[b],
                              fsems[b]).wait()

    def start_scatter(b):
        pltpu.async_copy(bufs.at[b], acc_sh.at[ibufs.at[b]], ssems[b], add=True)

    def wait_scatter(b):
        pltpu.make_async_copy(bufs.at[b], acc_sh.at[ibufs.at[b]],
                              ssems[b]).wait()

    for l in range(3):
        start_load(l, l)

    def body(j0, carry):
        for b in range(4):
            j = j0 * 4 + b
            wait_load(j, b)

            @pl.when(j >= 1)
            def _():
                wait_scatter((b - 1) % 4)

            start_scatter(b)

            @pl.when(j + 3 < _N_CHUNKS)
            def _():
                start_load(j + 3, (b + 3) % 4)

        return carry

    lax.fori_loop(0, (_N_CHUNKS - 1) // 4, body, 0)
    wait_load(_N_CHUNKS - 1, 0)
    wait_scatter(3)
    start_scatter(0)
    wait_scatter(0)
    plsc.subcore_barrier()

    pltpu.sync_copy(acc_sh.at[pl.ds(mol_base, _MOLS_PER_TILE)],
                    out_hbm.at[c, pl.ds(mol_base, _MOLS_PER_TILE)])


_ROWS_BLK = 1000


def _mlp_body(p_ref, w1_ref, b1_ref, w2_ref, b2_ref, out_ref, fp_ref):
    fp = p_ref[0] + p_ref[1]
    fp_ref[...] = fp
    h = jnp.dot(fp, w1_ref[...], preferred_element_type=jnp.float32) + b1_ref[...]
    sp = jnp.maximum(h, 0.0) + jnp.log(1.0 + jnp.exp(-jnp.abs(h))) - _LOG2
    out_ref[...] = jnp.sum(sp * w2_ref[...], axis=1, keepdims=True) + b2_ref[...]


def _mlp_tc(partials, W1, b1r, w2r, b2r):
    grid = (_N_MOLS // _ROWS_BLK,)
    return pl.pallas_call(
        _mlp_body,
        grid=grid,
        in_specs=[
            pl.BlockSpec((_NC, _ROWS_BLK, _FEAT), lambda i: (0, i, 0)),
            pl.BlockSpec((_FEAT, _HID), lambda i: (0, 0)),
            pl.BlockSpec((1, _HID), lambda i: (0, 0)),
            pl.BlockSpec((1, _HID), lambda i: (0, 0)),
            pl.BlockSpec((1, 1), lambda i: (0, 0)),
        ],
        out_specs=[
            pl.BlockSpec((_ROWS_BLK, 1), lambda i: (i, 0)),
            pl.BlockSpec((_ROWS_BLK, _FEAT), lambda i: (i, 0)),
        ],
        out_shape=[
            jax.ShapeDtypeStruct((_N_MOLS, 1), jnp.float32),
            jax.ShapeDtypeStruct((_N_MOLS, _FEAT), jnp.float32),
        ],
    )(partials, W1, b1r, w2r, b2r)


def kernel(feats, segment_ids, W1, b1, W2, b2):
    ids = segment_ids.astype(jnp.int32)
    partials = _segsum_sc(feats, ids)
    out2d, mol_fp = _mlp_tc(partials, W1, b1.reshape(1, _HID),
                            W2.reshape(1, _HID), b2.reshape(1, 1))
    return out2d.reshape(-1), mol_fp


# speedup vs baseline: 1.1801x; 1.0095x over previous
import functools

import jax
import jax.numpy as jnp
from jax import lax
from jax.experimental import pallas as pl
from jax.experimental.pallas import tpu as pltpu
from jax.experimental.pallas import tpu_sc as plsc

_N_ATOMS = 320000
_N_MOLS = 10000
_FEAT = 128
_HID = 64

_NC = 2
_NS = 16
_NW = _NC * _NS
_ATOMS_PER_TILE = _N_ATOMS // _NW
_CHUNK = 80
_N_CHUNKS = _ATOMS_PER_TILE // _CHUNK
_MOLS_PAD = 10240
_MOLS_PER_TILE = _MOLS_PAD // _NS

_LOG2 = 0.6931471805599453


@functools.partial(
    pl.kernel,
    mesh=plsc.VectorSubcoreMesh(core_axis_name="c", subcore_axis_name="s"),
    out_type=jax.ShapeDtypeStruct((_NC, _MOLS_PAD, _FEAT), jnp.float32),
    scratch_types=[
        pltpu.VMEM((4, _CHUNK), jnp.int32),
        pltpu.VMEM((4, _CHUNK, _FEAT), jnp.float32),
        pltpu.VMEM_SHARED((_MOLS_PAD, _FEAT), jnp.float32),
        pltpu.SemaphoreType.DMA,
        pltpu.SemaphoreType.DMA,
        pltpu.SemaphoreType.DMA,
        pltpu.SemaphoreType.DMA,
        pltpu.SemaphoreType.DMA,
        pltpu.SemaphoreType.DMA,
        pltpu.SemaphoreType.DMA,
        pltpu.SemaphoreType.DMA,
        pltpu.SemaphoreType.DMA,
        pltpu.SemaphoreType.DMA,
        pltpu.SemaphoreType.DMA,
        pltpu.SemaphoreType.DMA,
    ],
)
def _segsum_sc(feats_hbm, ids_hbm, out_hbm, ibufs, bufs, acc_sh,
               fsem0, fsem1, fsem2, fsem3, isem0, isem1, isem2, isem3,
               ssem0, ssem1, ssem2, ssem3):
    c = lax.axis_index("c")
    s = lax.axis_index("s")
    w = c * _NS + s
    tile_base = w * _ATOMS_PER_TILE
    fsems = (fsem0, fsem1, fsem2, fsem3)
    isems = (isem0, isem1, isem2, isem3)
    ssems = (ssem0, ssem1, ssem2, ssem3)

    def start_load(i, b):
        a = tile_base + i * _CHUNK
        pltpu.async_copy(ids_hbm.at[pl.ds(a, _CHUNK)], ibufs.at[b], isems[b])
        pltpu.async_copy(feats_hbm.at[pl.ds(a, _CHUNK)], bufs.at[b], fsems[b])

    start_load(1, 1)
    start_load(2, 2)

    zeros16 = jnp.zeros((16,), jnp.float32)

    def zb(i, carry):
        bufs[0, i // (_FEAT // 16), pl.ds((i % (_FEAT // 16)) * 16, 16)] = zeros16
        return carry

    lax.fori_loop(0, _CHUNK * (_FEAT // 16), zb, 0)

    mol_base = s * _MOLS_PER_TILE
    for j in range(_MOLS_PER_TILE // _CHUNK):
        pltpu.sync_copy(bufs.at[0], acc_sh.at[pl.ds(mol_base + j * _CHUNK, _CHUNK)])
    start_load(0, 0)
    plsc.subcore_barrier()

    def wait_load(i, b):
        a = tile_base + i * _CHUNK
        pltpu.make_async_copy(ids_hbm.at[pl.ds(a, _CHUNK)], ibufs.at[b],
                              isems[b]).wait()
        pltpu.make_async_copy(feats_hbm.at[pl.ds(a, _CHUNK)], bufs.at[b],
                              fsems[b]).wait()

    def start_scatter(b):
        pltpu.async_copy(bufs.at[b], acc_sh.at[ibufs.at[b]], ssems[b], add=True)

    def wait_scatter(b):
        pltpu.make_async_copy(bufs.at[b], acc_sh.at[ibufs.at[b]],
                              ssems[b]).wait()

    def body(j0, carry):
        for b in range(4):
            j = j0 * 4 + b
            wait_load(j, b)

            @pl.when(j >= 1)
            def _():
                wait_scatter((b - 1) % 4)

            start_scatter(b)

            @pl.when(j + 3 < _N_CHUNKS)
            def _():
                start_load(j + 3, (b + 3) % 4)

        return carry

    lax.fori_loop(0, (_N_CHUNKS - 1) // 4, body, 0)
    wait_load(_N_CHUNKS - 1, 0)
    wait_scatter(3)
    start_scatter(0)
    wait_scatter(0)
    plsc.subcore_barrier()

    pltpu.sync_copy(acc_sh.at[pl.ds(mol_base, _MOLS_PER_TILE)],
                    out_hbm.at[c, pl.ds(mol_base, _MOLS_PER_TILE)])


_ROWS_BLK = 1000


def _mlp_body(p_ref, w1_ref, b1_ref, w2_ref, b2_ref, out_ref, fp_ref):
    fp = p_ref[0] + p_ref[1]
    fp_ref[...] = fp
    h = jnp.dot(fp, w1_ref[...], preferred_element_type=jnp.float32) + b1_ref[...]
    sp = jnp.maximum(h, 0.0) + jnp.log(1.0 + jnp.exp(-jnp.abs(h))) - _LOG2
    out_ref[...] = jnp.sum(sp * w2_ref[...], axis=1, keepdims=True) + b2_ref[...]


def _mlp_tc(partials, W1, b1r, w2r, b2r):
    grid = (_N_MOLS // _ROWS_BLK,)
    return pl.pallas_call(
        _mlp_body,
        grid=grid,
        in_specs=[
            pl.BlockSpec((_NC, _ROWS_BLK, _FEAT), lambda i: (0, i, 0)),
            pl.BlockSpec((_FEAT, _HID), lambda i: (0, 0)),
            pl.BlockSpec((1, _HID), lambda i: (0, 0)),
            pl.BlockSpec((1, _HID), lambda i: (0, 0)),
            pl.BlockSpec((1, 1), lambda i: (0, 0)),
        ],
        out_specs=[
            pl.BlockSpec((_ROWS_BLK, 1), lambda i: (i, 0)),
            pl.BlockSpec((_ROWS_BLK, _FEAT), lambda i: (i, 0)),
        ],
        out_shape=[
            jax.ShapeDtypeStruct((_N_MOLS, 1), jnp.float32),
            jax.ShapeDtypeStruct((_N_MOLS, _FEAT), jnp.float32),
        ],
    )(partials, W1, b1r, w2r, b2r)


def kernel(feats, segment_ids, W1, b1, W2, b2):
    ids = segment_ids.astype(jnp.int32)
    partials = _segsum_sc(feats, ids)
    out2d, mol_fp = _mlp_tc(partials, W1, b1.reshape(1, _HID),
                            W2.reshape(1, _HID), b2.reshape(1, 1))
    return out2d.reshape(-1), mol_fp
